# baseline (speedup 1.0000x reference)
import functools

import jax
import jax.numpy as jnp
from jax.experimental import pallas as pl
from jax.experimental.pallas import tpu as pltpu


def _onehot_body(rows_ref, out_ref, raw_ref, *, n_blk, n_classes, n_batch):
    j = pl.program_id(0)
    n_j = pl.num_programs(0)

    @pl.when(j == 0)
    def _prep_first():
        raw_ref[pl.ds(0, n_blk), :] = jnp.transpose(
            rows_ref[:, pl.ds(0, n_blk)], (1, 0))

    nxt = jax.lax.rem(j + 1, n_j)
    raw_ref[pl.ds(nxt * n_blk, n_blk), :] = jnp.transpose(
        rows_ref[:, pl.ds(nxt * n_blk, n_blk)], (1, 0))

    blk = raw_ref[pl.ds(j * n_blk, n_blk), :]
    nxt_row = raw_ref[pl.ds(nxt * n_blk, 8), :][0:1]
    nxt_val = jnp.where(j == n_j - 1, blk[-1:, :] + 1, nxt_row)
    shifted = jnp.concatenate([blk[1:, :], nxt_val], axis=0)
    shifted = jnp.where(shifted > n_classes - 1, 0, shifted)
    iota = jax.lax.broadcasted_iota(jnp.int32, (n_blk, n_classes), 1)
    for b in range(n_batch):
        col = shifted[:, b:b + 1]
        out_ref[b] = jnp.where(col == iota, jnp.float32(1.0), jnp.float32(0.0))


def kernel(inputs):
    B, T = inputs.shape
    K = 256
    N = 128
    C = T // N
    return pl.pallas_call(
        functools.partial(_onehot_body, n_blk=N, n_classes=K, n_batch=B),
        grid=(C,),
        in_specs=[pl.BlockSpec((B, T), lambda j: (0, 0))],
        out_specs=pl.BlockSpec((B, N, K), lambda j: (0, j, 0)),
        out_shape=jax.ShapeDtypeStruct((B, T, K), jnp.float32),
        scratch_shapes=[pltpu.VMEM((T, B), jnp.int32)],
        compiler_params=pltpu.CompilerParams(
            dimension_semantics=("arbitrary",),
        ),
    )(inputs.astype(jnp.int32))


# speedup vs baseline: 1.0419x; 1.0419x over previous
import functools

import jax
import jax.numpy as jnp
from jax.experimental import pallas as pl
from jax.experimental.pallas import tpu as pltpu


def _onehot_body(rows_ref, out_ref, raw_ref, *, n_blk, n_classes, n_batch,
                 n_pieces, piece):
    j = pl.program_id(0)
    n_j = pl.num_programs(0)

    for p in range(n_pieces):
        @pl.when(j == p)
        def _prep(p=p):
            raw_ref[pl.ds(p * piece, piece), :] = jnp.transpose(
                rows_ref[:, pl.ds(p * piece, piece)], (1, 0))

    blk = raw_ref[pl.ds(j * n_blk, n_blk), :]
    nxt = jax.lax.rem(j + 1, n_j)
    nxt_row = raw_ref[pl.ds(nxt * n_blk, 8), :][0:1]
    nxt_val = jnp.where(j == n_j - 1, blk[-1:, :] + 1, nxt_row)
    shifted = jnp.concatenate([blk[1:, :], nxt_val], axis=0)
    shifted = jnp.where(shifted > n_classes - 1, 0, shifted)
    iota = jax.lax.broadcasted_iota(jnp.int32, (n_blk, n_classes), 1)
    for b in range(n_batch):
        col = shifted[:, b:b + 1]
        out_ref[b] = jnp.where(col == iota, jnp.float32(1.0), jnp.float32(0.0))


def kernel(inputs):
    B, T = inputs.shape
    K = 256
    N = 128
    C = T // N
    P = 4
    return pl.pallas_call(
        functools.partial(_onehot_body, n_blk=N, n_classes=K, n_batch=B,
                          n_pieces=P, piece=T // P),
        grid=(C,),
        in_specs=[pl.BlockSpec((B, T), lambda j: (0, 0))],
        out_specs=pl.BlockSpec((B, N, K), lambda j: (0, j, 0)),
        out_shape=jax.ShapeDtypeStruct((B, T, K), jnp.float32),
        scratch_shapes=[pltpu.VMEM((T, B), jnp.int32)],
        compiler_params=pltpu.CompilerParams(
            dimension_semantics=("arbitrary",),
        ),
    )(inputs.astype(jnp.int32))


# speedup vs baseline: 1.1568x; 1.1103x over previous
import functools

import jax
import jax.numpy as jnp
from jax.experimental import pallas as pl
from jax.experimental.pallas import tpu as pltpu


def _onehot_body(rows_ref, out_ref, raw_ref, *, n_blk, n_classes, n_batch,
                 n_pieces, piece):
    j = pl.program_id(0)
    n_j = pl.num_programs(0)

    for p in range(n_pieces):
        @pl.when(j == p)
        def _prep(p=p):
            raw_ref[pl.ds(p * piece, piece), :] = jnp.transpose(
                rows_ref[:, pl.ds(p * piece, piece)], (1, 0))

    blk = raw_ref[pl.ds(j * n_blk, n_blk), :]
    nxt = jax.lax.rem(j + 1, n_j)
    nxt_row = raw_ref[pl.ds(nxt * n_blk, 8), :][0:1]
    nxt_val = jnp.where(j == n_j - 1, blk[-1:, :] + 1, nxt_row)
    shifted = jnp.concatenate([blk[1:, :], nxt_val], axis=0)
    shifted = jnp.where(shifted > n_classes - 1, 0, shifted)
    iota = jax.lax.broadcasted_iota(jnp.int32, (n_blk, n_classes), 1)
    for b in range(n_batch):
        col = shifted[:, b:b + 1]
        out_ref[b] = jnp.where(col == iota, jnp.float32(1.0), jnp.float32(0.0))


def kernel(inputs):
    B, T = inputs.shape
    K = 256
    N = 256
    C = T // N
    P = 4
    return pl.pallas_call(
        functools.partial(_onehot_body, n_blk=N, n_classes=K, n_batch=B,
                          n_pieces=P, piece=T // P),
        grid=(C,),
        in_specs=[pl.BlockSpec((B, T), lambda j: (0, 0))],
        out_specs=pl.BlockSpec((B, N, K), lambda j: (0, j, 0)),
        out_shape=jax.ShapeDtypeStruct((B, T, K), jnp.float32),
        scratch_shapes=[pltpu.VMEM((T, B), jnp.int32)],
        compiler_params=pltpu.CompilerParams(
            dimension_semantics=("arbitrary",),
        ),
    )(inputs.astype(jnp.int32))
